# baseline (speedup 1.0000x reference)
import dataclasses
import functools

import jax
import jax.numpy as jnp
from jax import lax
from jax.experimental import pallas as pl
from jax.experimental.pallas import tpu as pltpu
from jax.experimental.pallas import tpu_sc as plsc

_B = 16384
_D = 128
_GAMMA = 12.0
_NW = 32
_BPW = _B // _NW
_CH = 128
_NCHUNK = _BPW // _CH


def _score_body(h_hbm, r_hbm, t_hbm, ent_hbm, rel_hbm, out_hbm,
                hidx, ridx, tidx, hbuf, rbuf, tbuf, outv, sem):
    nc = plsc.get_sparse_core_info().num_cores
    wid = lax.axis_index("s") * nc + lax.axis_index("c")

    pltpu.sync_copy(h_hbm.at[wid], hidx)
    pltpu.sync_copy(r_hbm.at[wid], ridx)
    pltpu.sync_copy(t_hbm.at[wid], tidx)

    lanes = lax.iota(jnp.int32, 16)

    @pl.loop(0, _NCHUNK)
    def _chunk(c):
        pltpu.sync_copy(ent_hbm.at[hidx.at[c]], hbuf)
        pltpu.sync_copy(rel_hbm.at[ridx.at[c]], rbuf)
        pltpu.sync_copy(ent_hbm.at[tidx.at[c]], tbuf)

        @pl.loop(0, _CH // 16)
        def _group(g):
            rows = lanes + g * 16

            def dbody(d, acc):
                cold = jnp.full((16,), 0, jnp.int32) + d
                hv = plsc.load_gather(hbuf, [rows, cold])
                rv = plsc.load_gather(rbuf, [rows, cold])
                tv = plsc.load_gather(tbuf, [rows, cold])
                return acc + jnp.abs(hv + rv - tv)

            acc = lax.fori_loop(0, _D, dbody, jnp.zeros((16,), jnp.float32))
            outv[pl.ds(c * _CH + g * 16, 16)] = _GAMMA - acc

    pltpu.sync_copy(outv, out_hbm.at[pl.ds(wid * _BPW, _BPW)])


_mesh = plsc.VectorSubcoreMesh(core_axis_name="c", subcore_axis_name="s")

_cp = pltpu.CompilerParams()
if "needs_layout_passes" in pltpu.CompilerParams.__dataclass_fields__:
    _cp = dataclasses.replace(_cp, needs_layout_passes=False)

_score_kernel = functools.partial(
    pl.kernel,
    mesh=_mesh,
    compiler_params=_cp,
    out_type=jax.ShapeDtypeStruct((_B,), jnp.float32),
    scratch_types=[
        pltpu.VMEM((_NCHUNK, _CH), jnp.int32),
        pltpu.VMEM((_NCHUNK, _CH), jnp.int32),
        pltpu.VMEM((_NCHUNK, _CH), jnp.int32),
        pltpu.VMEM((_CH, _D), jnp.float32),
        pltpu.VMEM((_CH, _D), jnp.float32),
        pltpu.VMEM((_CH, _D), jnp.float32),
        pltpu.VMEM((_BPW,), jnp.float32),
        pltpu.SemaphoreType.DMA,
    ],
)(_score_body)


@jax.jit
def kernel(sample, entity_embedding, relation_embedding):
    idx = sample.T.reshape(3, _NW, _NCHUNK, _CH)
    scores = _score_kernel(idx[0], idx[1], idx[2],
                           entity_embedding, relation_embedding)
    return scores.reshape(_B, 1)


# speedup vs baseline: 1.1991x; 1.1991x over previous
import dataclasses
import functools

import jax
import jax.numpy as jnp
from jax import lax
from jax.experimental import pallas as pl
from jax.experimental.pallas import tpu as pltpu
from jax.experimental.pallas import tpu_sc as plsc

_B = 16384
_D = 128
_GAMMA = 12.0
_NW = 32
_BPW = _B // _NW
_CH = 128
_NCHUNK = _BPW // _CH
_UNROLL = 8


def _score_body(h_hbm, r_hbm, t_hbm, ent_hbm, rel_hbm, out_hbm,
                hidx, ridx, tidx,
                hbuf0, rbuf0, tbuf0, hbuf1, rbuf1, tbuf1,
                outv, sem0, sem1):
    nc = plsc.get_sparse_core_info().num_cores
    wid = lax.axis_index("s") * nc + lax.axis_index("c")

    pltpu.sync_copy(h_hbm.at[wid], hidx)
    pltpu.sync_copy(r_hbm.at[wid], ridx)
    pltpu.sync_copy(t_hbm.at[wid], tidx)

    bufs = ((hbuf0, rbuf0, tbuf0), (hbuf1, rbuf1, tbuf1))
    sems = (sem0, sem1)
    lanes = lax.iota(jnp.int32, 16)

    def start(c, slot):
        hb, rb, tb = bufs[slot]
        sem = sems[slot]
        return (
            pltpu.async_copy(ent_hbm.at[hidx.at[c]], hb, sem),
            pltpu.async_copy(rel_hbm.at[ridx.at[c]], rb, sem),
            pltpu.async_copy(ent_hbm.at[tidx.at[c]], tb, sem),
        )

    def compute(c, slot):
        hb, rb, tb = bufs[slot]

        @pl.loop(0, _CH // 16)
        def _group(g):
            rows = lanes + g * 16

            def dbody(j, acc):
                d0 = j * _UNROLL
                for k in range(_UNROLL):
                    cold = jnp.full((16,), 0, jnp.int32) + (d0 + k)
                    hv = plsc.load_gather(hb, [rows, cold])
                    rv = plsc.load_gather(rb, [rows, cold])
                    tv = plsc.load_gather(tb, [rows, cold])
                    acc = acc + jnp.abs(hv + rv - tv)
                return acc

            acc = lax.fori_loop(0, _D // _UNROLL, dbody,
                                jnp.zeros((16,), jnp.float32))
            outv[pl.ds(c * _CH + g * 16, 16)] = _GAMMA - acc

    handles = [None, None]
    handles[0] = start(0, 0)
    for c in range(_NCHUNK):
        if c + 1 < _NCHUNK:
            handles[(c + 1) % 2] = start(c + 1, (c + 1) % 2)
        for h in handles[c % 2]:
            h.wait()
        compute(c, c % 2)

    pltpu.sync_copy(outv, out_hbm.at[pl.ds(wid * _BPW, _BPW)])


_mesh = plsc.VectorSubcoreMesh(core_axis_name="c", subcore_axis_name="s")

_cp = pltpu.CompilerParams()
if "needs_layout_passes" in pltpu.CompilerParams.__dataclass_fields__:
    _cp = dataclasses.replace(_cp, needs_layout_passes=False)

_score_kernel = functools.partial(
    pl.kernel,
    mesh=_mesh,
    compiler_params=_cp,
    out_type=jax.ShapeDtypeStruct((_B,), jnp.float32),
    scratch_types=[
        pltpu.VMEM((_NCHUNK, _CH), jnp.int32),
        pltpu.VMEM((_NCHUNK, _CH), jnp.int32),
        pltpu.VMEM((_NCHUNK, _CH), jnp.int32),
        pltpu.VMEM((_CH, _D), jnp.float32),
        pltpu.VMEM((_CH, _D), jnp.float32),
        pltpu.VMEM((_CH, _D), jnp.float32),
        pltpu.VMEM((_CH, _D), jnp.float32),
        pltpu.VMEM((_CH, _D), jnp.float32),
        pltpu.VMEM((_CH, _D), jnp.float32),
        pltpu.VMEM((_BPW,), jnp.float32),
        pltpu.SemaphoreType.DMA,
        pltpu.SemaphoreType.DMA,
    ],
)(_score_body)


@jax.jit
def kernel(sample, entity_embedding, relation_embedding):
    idx = sample.T.reshape(3, _NW, _NCHUNK, _CH)
    scores = _score_kernel(idx[0], idx[1], idx[2],
                           entity_embedding, relation_embedding)
    return scores.reshape(_B, 1)


# speedup vs baseline: 3.0516x; 2.5449x over previous
import dataclasses
import functools

import jax
import jax.numpy as jnp
from jax import lax
from jax.experimental import pallas as pl
from jax.experimental.pallas import tpu as pltpu
from jax.experimental.pallas import tpu_sc as plsc

_B = 16384
_D = 128
_GAMMA = 12.0
_NW = 32
_BPW = _B // _NW
_CH = 128
_NCHUNK = _BPW // _CH
_UNROLL = 8


def _score_body(h_hbm, r_hbm, t_hbm, ent_hbm, rel_hbm, out_hbm,
                hidx, ridx, tidx,
                hbuf0, rbuf0, tbuf0, hbuf1, rbuf1, tbuf1,
                tr, outv, sem0, sem1):
    nc = plsc.get_sparse_core_info().num_cores
    wid = lax.axis_index("s") * nc + lax.axis_index("c")

    pltpu.sync_copy(h_hbm.at[wid], hidx)
    pltpu.sync_copy(r_hbm.at[wid], ridx)
    pltpu.sync_copy(t_hbm.at[wid], tidx)

    bufs = ((hbuf0, rbuf0, tbuf0), (hbuf1, rbuf1, tbuf1))
    sems = (sem0, sem1)
    lanes = lax.iota(jnp.int32, 16)

    def start(c, slot):
        hb, rb, tb = bufs[slot]
        sem = sems[slot]
        return (
            pltpu.async_copy(ent_hbm.at[hidx.at[c]], hb, sem),
            pltpu.async_copy(rel_hbm.at[ridx.at[c]], rb, sem),
            pltpu.async_copy(ent_hbm.at[tidx.at[c]], tb, sem),
        )

    def compute(c, slot):
        hb, rb, tb = bufs[slot]

        @pl.loop(0, _CH // 16)
        def _group(g):
            base = g * 16
            for i in range(16):
                hrow = hb.at[base + i]
                rrow = rb.at[base + i]
                trow = tb.at[base + i]
                acc = None
                for cc in range(_D // 16):
                    sl = pl.ds(cc * 16, 16)
                    v = jnp.abs(hrow[sl] + rrow[sl] - trow[sl])
                    acc = v if acc is None else acc + v
                tr[i, :] = acc
            s = None
            for j in range(16):
                colj = plsc.load_gather(
                    tr, [lanes, jnp.full((16,), j, jnp.int32)])
                s = colj if s is None else s + colj
            outv[pl.ds(c * _CH + base, 16)] = _GAMMA - s

    handles = [None, None]
    handles[0] = start(0, 0)
    for c in range(_NCHUNK):
        if c + 1 < _NCHUNK:
            handles[(c + 1) % 2] = start(c + 1, (c + 1) % 2)
        for h in handles[c % 2]:
            h.wait()
        compute(c, c % 2)

    pltpu.sync_copy(outv, out_hbm.at[pl.ds(wid * _BPW, _BPW)])


_mesh = plsc.VectorSubcoreMesh(core_axis_name="c", subcore_axis_name="s")

_cp = pltpu.CompilerParams()
if "needs_layout_passes" in pltpu.CompilerParams.__dataclass_fields__:
    _cp = dataclasses.replace(_cp, needs_layout_passes=False)

_score_kernel = functools.partial(
    pl.kernel,
    mesh=_mesh,
    compiler_params=_cp,
    out_type=jax.ShapeDtypeStruct((_B,), jnp.float32),
    scratch_types=[
        pltpu.VMEM((_NCHUNK, _CH), jnp.int32),
        pltpu.VMEM((_NCHUNK, _CH), jnp.int32),
        pltpu.VMEM((_NCHUNK, _CH), jnp.int32),
        pltpu.VMEM((_CH, _D), jnp.float32),
        pltpu.VMEM((_CH, _D), jnp.float32),
        pltpu.VMEM((_CH, _D), jnp.float32),
        pltpu.VMEM((_CH, _D), jnp.float32),
        pltpu.VMEM((_CH, _D), jnp.float32),
        pltpu.VMEM((_CH, _D), jnp.float32),
        pltpu.VMEM((16, 16), jnp.float32),
        pltpu.VMEM((_BPW,), jnp.float32),
        pltpu.SemaphoreType.DMA,
        pltpu.SemaphoreType.DMA,
    ],
)(_score_body)


@jax.jit
def kernel(sample, entity_embedding, relation_embedding):
    idx = sample.T.reshape(3, _NW, _NCHUNK, _CH)
    scores = _score_kernel(idx[0], idx[1], idx[2],
                           entity_embedding, relation_embedding)
    return scores.reshape(_B, 1)


# speedup vs baseline: 3.3048x; 1.0830x over previous
import dataclasses
import functools

import jax
import jax.numpy as jnp
from jax import lax
from jax.experimental import pallas as pl
from jax.experimental.pallas import tpu as pltpu
from jax.experimental.pallas import tpu_sc as plsc

_B = 16384
_D = 128
_GAMMA = 12.0
_NW = 32
_BPW = _B // _NW
_CH = 128
_NCHUNK = _BPW // _CH
_UNROLL = 8
_WAVE = 4


def _score_body(h_hbm, r_hbm, t_hbm, ent_hbm, rel_hbm, out_hbm,
                hidx, ridx, tidx,
                hbuf0, rbuf0, tbuf0, hbuf1, rbuf1, tbuf1,
                tr, outv, sem0, sem1):
    nc = plsc.get_sparse_core_info().num_cores
    wid = lax.axis_index("s") * nc + lax.axis_index("c")

    pltpu.sync_copy(h_hbm.at[wid], hidx)
    pltpu.sync_copy(r_hbm.at[wid], ridx)
    pltpu.sync_copy(t_hbm.at[wid], tidx)

    bufs = ((hbuf0, rbuf0, tbuf0), (hbuf1, rbuf1, tbuf1))
    sems = (sem0, sem1)
    lanes = lax.iota(jnp.int32, 16)

    def start(c, slot):
        hb, rb, tb = bufs[slot]
        sem = sems[slot]
        return (
            pltpu.async_copy(ent_hbm.at[hidx.at[c]], hb, sem),
            pltpu.async_copy(rel_hbm.at[ridx.at[c]], rb, sem),
            pltpu.async_copy(ent_hbm.at[tidx.at[c]], tb, sem),
        )

    def compute(c, slot):
        hb, rb, tb = bufs[slot]

        @pl.loop(0, _CH // 16)
        def _group(g):
            base = g * 16
            for w in range(16 // _WAVE):
                accs = []
                for i in range(w * _WAVE, (w + 1) * _WAVE):
                    hrow = hb.at[base + i]
                    rrow = rb.at[base + i]
                    trow = tb.at[base + i]
                    vs = []
                    for cc in range(_D // 16):
                        sl = pl.ds(cc * 16, 16)
                        vs.append(jnp.abs(hrow[sl] + rrow[sl] - trow[sl]))
                    while len(vs) > 1:
                        vs = [vs[k] + vs[k + 1] for k in range(0, len(vs), 2)]
                    accs.append(vs[0])
                for i, acc in enumerate(accs):
                    tr[w * _WAVE + i, :] = acc
            cols = [plsc.load_gather(tr, [lanes, jnp.full((16,), j, jnp.int32)])
                    for j in range(16)]
            while len(cols) > 1:
                cols = [cols[k] + cols[k + 1] for k in range(0, len(cols), 2)]
            outv[pl.ds(c * _CH + base, 16)] = _GAMMA - cols[0]

    handles = [None, None]
    handles[0] = start(0, 0)
    for c in range(_NCHUNK):
        if c + 1 < _NCHUNK:
            handles[(c + 1) % 2] = start(c + 1, (c + 1) % 2)
        for h in handles[c % 2]:
            h.wait()
        compute(c, c % 2)

    pltpu.sync_copy(outv, out_hbm.at[pl.ds(wid * _BPW, _BPW)])


_mesh = plsc.VectorSubcoreMesh(core_axis_name="c", subcore_axis_name="s")

_cp = pltpu.CompilerParams()
if "needs_layout_passes" in pltpu.CompilerParams.__dataclass_fields__:
    _cp = dataclasses.replace(_cp, needs_layout_passes=False)

_score_kernel = functools.partial(
    pl.kernel,
    mesh=_mesh,
    compiler_params=_cp,
    out_type=jax.ShapeDtypeStruct((_B,), jnp.float32),
    scratch_types=[
        pltpu.VMEM((_NCHUNK, _CH), jnp.int32),
        pltpu.VMEM((_NCHUNK, _CH), jnp.int32),
        pltpu.VMEM((_NCHUNK, _CH), jnp.int32),
        pltpu.VMEM((_CH, _D), jnp.float32),
        pltpu.VMEM((_CH, _D), jnp.float32),
        pltpu.VMEM((_CH, _D), jnp.float32),
        pltpu.VMEM((_CH, _D), jnp.float32),
        pltpu.VMEM((_CH, _D), jnp.float32),
        pltpu.VMEM((_CH, _D), jnp.float32),
        pltpu.VMEM((16, 16), jnp.float32),
        pltpu.VMEM((_BPW,), jnp.float32),
        pltpu.SemaphoreType.DMA,
        pltpu.SemaphoreType.DMA,
    ],
)(_score_body)


@jax.jit
def kernel(sample, entity_embedding, relation_embedding):
    idx = sample.T.reshape(3, _NW, _NCHUNK, _CH)
    scores = _score_kernel(idx[0], idx[1], idx[2],
                           entity_embedding, relation_embedding)
    return scores.reshape(_B, 1)
